# baseline (speedup 1.0000x reference)
import jax
import jax.numpy as jnp
from jax import lax
from jax.experimental import pallas as pl
from jax.experimental.pallas import tpu as pltpu
from jax.experimental.pallas import tpu_sc as plsc

D_MODEL = 768
NUM_INDICES = 4 * 8192

_info = plsc.get_sparse_core_info()
_NC, _NS = _info.num_cores, _info.num_subcores
_NW = _NC * _NS
_PER_W = NUM_INDICES // _NW
_CHUNK = 16
_NCHUNK = _PER_W // _CHUNK
_NBUF = 8
_AHEAD = 4


def _gather_body(table_hbm, idx_hbm, out_hbm, idx_v, *refs):
    bufs = refs[:_NBUF]
    sem_idx = refs[_NBUF]
    gsems = refs[_NBUF + 1:2 * _NBUF + 1]
    osems = refs[2 * _NBUF + 1:]

    wid = lax.axis_index("s") * _NC + lax.axis_index("c")
    base = wid * _PER_W

    pltpu.async_copy(idx_hbm.at[wid], idx_v, sem_idx).wait()

    def gather(j):
        return pltpu.make_async_copy(
            table_hbm.at[idx_v.at[j]], bufs[j % _NBUF], gsems[j % _NBUF])

    def scatter(j):
        return pltpu.make_async_copy(
            bufs[j % _NBUF], out_hbm.at[pl.ds(base + j * _CHUNK, _CHUNK)],
            osems[j % _NBUF])

    for j in range(_AHEAD):
        gather(j).start()
    for j in range(_NCHUNK):
        gather(j).wait()
        scatter(j).start()
        if j + _AHEAD < _NCHUNK:
            if j >= _NBUF - _AHEAD:
                scatter(j - (_NBUF - _AHEAD)).wait()
            gather(j + _AHEAD).start()
    for j in range(_NCHUNK - 2 * _AHEAD, _NCHUNK):
        scatter(j).wait()


def kernel(indices, emb_weight):
    b, l = indices.shape
    idx_flat = indices.reshape(_NW, _NCHUNK, _CHUNK).astype(jnp.int32)

    mesh = plsc.VectorSubcoreMesh(core_axis_name="c", subcore_axis_name="s")
    run = pl.kernel(
        _gather_body,
        mesh=mesh,
        out_type=jax.ShapeDtypeStruct((NUM_INDICES, D_MODEL), jnp.float32),
        scratch_types=(
            [pltpu.VMEM((_NCHUNK, _CHUNK), jnp.int32)]
            + [pltpu.VMEM((_CHUNK, D_MODEL), jnp.float32)] * _NBUF
            + [pltpu.SemaphoreType.DMA] * (1 + 2 * _NBUF)
        ),
    )
    out = run(emb_weight, idx_flat)
    return out.reshape(b, l, D_MODEL)


# speedup vs baseline: 1.0037x; 1.0037x over previous
import jax
import jax.numpy as jnp
from jax import lax
from jax.experimental import pallas as pl
from jax.experimental.pallas import tpu as pltpu
from jax.experimental.pallas import tpu_sc as plsc

D_MODEL = 768
NUM_INDICES = 4 * 8192

_info = plsc.get_sparse_core_info()
_NC, _NS = _info.num_cores, _info.num_subcores
_NW = _NC * _NS
_PER_W = NUM_INDICES // _NW
_CHUNK = 128
_NCHUNK = _PER_W // _CHUNK


def _gather_body(table_hbm, idx_hbm, out_hbm, idx_v, rows_v, sem_idx, sem_g):
    wid = lax.axis_index("s") * _NC + lax.axis_index("c")
    base = wid * _PER_W

    pltpu.async_copy(idx_hbm.at[wid], idx_v, sem_idx).wait()

    def step(j, carry):
        pltpu.async_copy(table_hbm.at[idx_v.at[j]], rows_v, sem_g).wait()
        pltpu.sync_copy(rows_v, out_hbm.at[pl.ds(base + j * _CHUNK, _CHUNK)])
        return carry

    lax.fori_loop(0, _NCHUNK, step, 0)


def kernel(indices, emb_weight):
    b, l = indices.shape
    idx_flat = indices.reshape(_NW, _NCHUNK, _CHUNK).astype(jnp.int32)

    mesh = plsc.VectorSubcoreMesh(core_axis_name="c", subcore_axis_name="s")
    run = pl.kernel(
        _gather_body,
        mesh=mesh,
        out_type=jax.ShapeDtypeStruct((NUM_INDICES, D_MODEL), jnp.float32),
        scratch_types=[
            pltpu.VMEM((_NCHUNK, _CHUNK), jnp.int32),
            pltpu.VMEM((_CHUNK, D_MODEL), jnp.float32),
            pltpu.SemaphoreType.DMA,
            pltpu.SemaphoreType.DMA,
        ],
    )
    out = run(emb_weight, idx_flat)
    return out.reshape(b, l, D_MODEL)
